# baseline (speedup 1.0000x reference)
import functools

import jax
import jax.numpy as jnp
from jax import lax
from jax.experimental import pallas as pl
from jax.experimental.pallas import tpu as pltpu
from jax.experimental.pallas import tpu_sc as plsc

D_MODEL = 64
NUM_CORES = 2
NUM_SUBCORES = 16
NUM_WORKERS = NUM_CORES * NUM_SUBCORES
CHUNK = 256
NBUF = 4
LANES = 16
GROUP = 512

REPACK_COLS = 2048
REPACK_GRID = 245
HALF_ROWS = REPACK_COLS * REPACK_GRID
TABLE_ROWS = 2 * HALF_ROWS


def _eye(n):
    r = lax.broadcasted_iota(jnp.int32, (n, n), 0)
    c = lax.broadcasted_iota(jnp.int32, (n, n), 1)
    return jnp.where(r == c, 1.0, 0.0).astype(jnp.float32)


def _mxu_t(x):
    return lax.dot_general(x, _eye(x.shape[0]), (((0,), (0,)), ((), ())),
                           preferred_element_type=jnp.float32)


def _repack_body(left_ref, right_ref, out_ref):
    out_ref[:, 0:D_MODEL] = _mxu_t(left_ref[...])
    out_ref[:, D_MODEL:2 * D_MODEL] = _mxu_t(right_ref[...])


@functools.lru_cache(maxsize=None)
def _make_repack(V: int):
    return pl.pallas_call(
        _repack_body,
        grid=(REPACK_GRID,),
        in_specs=[
            pl.BlockSpec((D_MODEL, REPACK_COLS), lambda i: (0, i)),
            pl.BlockSpec((D_MODEL, REPACK_COLS),
                         lambda i: (0, jnp.minimum(i + REPACK_GRID,
                                                   (V - 1) // REPACK_COLS))),
        ],
        out_specs=pl.BlockSpec((REPACK_COLS, 2 * D_MODEL), lambda i: (i, 0)),
        out_shape=jax.ShapeDtypeStruct((HALF_ROWS, 2 * D_MODEL), jnp.float32),
    )


def _unpack_body(in_ref, out_ref):
    blk = in_ref[...]
    out_ref[0, :, 0:GROUP // 2] = _mxu_t(blk[:, 0:D_MODEL])
    out_ref[0, :, GROUP // 2:GROUP] = _mxu_t(blk[:, D_MODEL:2 * D_MODEL])


@functools.lru_cache(maxsize=None)
def _make_unpack(B: int, H: int):
    gb = B // GROUP
    return pl.pallas_call(
        _unpack_body,
        grid=(H, gb),
        in_specs=[pl.BlockSpec((GROUP // 2, 2 * D_MODEL),
                               lambda h, bb: (h * gb + bb, 0))],
        out_specs=pl.BlockSpec((1, D_MODEL, GROUP), lambda h, bb: (h, 0, bb)),
        out_shape=jax.ShapeDtypeStruct((H, D_MODEL, B), jnp.float32),
    )


@functools.lru_cache(maxsize=None)
def _make_lookup(B: int):
    assert B % (NUM_WORKERS * CHUNK * NBUF) == 0
    b_per_w = B // NUM_WORKERS
    n_chunks = b_per_w // CHUNK
    n_rounds = n_chunks // NBUF
    mesh = plsc.VectorSubcoreMesh(
        core_axis_name="c", subcore_axis_name="s",
        num_cores=NUM_CORES, num_subcores=NUM_SUBCORES)

    @functools.partial(
        pl.kernel,
        out_type=jax.ShapeDtypeStruct((B, D_MODEL), jnp.float32),
        mesh=mesh,
        scratch_types=[
            pltpu.VMEM((b_per_w,), jnp.int32),
            pltpu.VMEM((b_per_w,), jnp.int32),
            pltpu.VMEM((NBUF, CHUNK, D_MODEL), jnp.float32),
        ] + [pltpu.SemaphoreType.DMA] * (2 * NBUF),
        compiler_params=pltpu.CompilerParams(
            use_tc_tiling_on_sc=False, needs_layout_passes=False),
    )
    def lookup(table_hbm, idx_hbm, out_hbm, idx_v, idx_p, rows_v, *sems):
        sem_g = sems[:NBUF]
        sem_o = sems[NBUF:]
        wid = lax.axis_index("s") * NUM_CORES + lax.axis_index("c")
        base = wid * b_per_w
        pltpu.sync_copy(idx_hbm.at[pl.ds(base, b_per_w)], idx_v)

        lane2 = 2 * lax.iota(jnp.int32, LANES)

        def remap(g, carry):
            for sub in range(2):
                for jc in range(GROUP // 2 // LANES):
                    p = g * GROUP + sub * (GROUP // 2) + jc * LANES
                    v = idx_v[pl.ds(p, LANES)]
                    v = jnp.where(v >= HALF_ROWS,
                                  2 * v - (TABLE_ROWS - 1), 2 * v)
                    pos = (g * GROUP + 2 * jc * LANES + sub) + lane2
                    plsc.store_scatter(idx_p, [pos], v)
            return carry

        lax.fori_loop(0, b_per_w // GROUP, remap, 0)

        def gather_desc(i, b):
            return pltpu.make_async_copy(
                table_hbm.at[idx_p.at[pl.ds(i * CHUNK, CHUNK)]],
                rows_v.at[b], sem_g[b])

        def out_desc(i, b):
            return pltpu.make_async_copy(
                rows_v.at[b], out_hbm.at[pl.ds(base + i * CHUNK, CHUNK)],
                sem_o[b])

        for b in range(NBUF):
            gather_desc(b, b).start()

        def body(r, carry):
            for b in range(NBUF):
                i = r * NBUF + b
                gather_desc(i, b).wait()
                out_desc(i, b).start()
            for b in range(NBUF):
                i = r * NBUF + b
                out_desc(i, b).wait()
                gather_desc(i + NBUF, b).start()
            return carry

        lax.fori_loop(0, n_rounds - 1, body, 0)

        r_last = n_rounds - 1
        for b in range(NBUF):
            i = r_last * NBUF + b
            gather_desc(i, b).wait()
            out_desc(i, b).start()
        for b in range(NBUF):
            out_desc(r_last * NBUF + b, b).wait()

    return lookup


@jax.jit
def kernel(x, weight):
    B, H = x.shape
    V = weight.shape[0]
    flat = jnp.transpose(x).reshape(B * H).astype(jnp.int32)
    wt = jnp.transpose(weight)
    table = _make_repack(V)(wt, wt).reshape(TABLE_ROWS, D_MODEL)
    out_lin = _make_lookup(B * H)(table, flat)
    out_t = _make_unpack(B, H)(out_lin.reshape(B * H // 2, 2 * D_MODEL))
    return jnp.transpose(out_t, (2, 0, 1))


# speedup vs baseline: 1.1170x; 1.1170x over previous
import functools

import jax
import jax.numpy as jnp
from jax import lax
from jax.experimental import pallas as pl
from jax.experimental.pallas import tpu as pltpu
from jax.experimental.pallas import tpu_sc as plsc

D_MODEL = 64
NUM_CORES = 2
NUM_SUBCORES = 16
NUM_WORKERS = NUM_CORES * NUM_SUBCORES
CHUNK = 256
NBUF = 4
LANES = 16
GROUP = 512

REPACK_COLS = 2048
REPACK_GRID = 245
HALF_ROWS = REPACK_COLS * REPACK_GRID
TABLE_ROWS = 2 * HALF_ROWS


def _repack_body(left_ref, right_ref, out_ref):
    lr = jnp.concatenate([left_ref[...], right_ref[...]], axis=0)
    out_ref[...] = jnp.transpose(lr, (1, 0))


@functools.lru_cache(maxsize=None)
def _make_repack(V: int):
    return pl.pallas_call(
        _repack_body,
        grid=(REPACK_GRID,),
        in_specs=[
            pl.BlockSpec((D_MODEL, REPACK_COLS), lambda i: (0, i)),
            pl.BlockSpec((D_MODEL, REPACK_COLS),
                         lambda i: (0, jnp.minimum(i + REPACK_GRID,
                                                   (V - 1) // REPACK_COLS))),
        ],
        out_specs=pl.BlockSpec((REPACK_COLS, 2 * D_MODEL), lambda i: (i, 0)),
        out_shape=jax.ShapeDtypeStruct((HALF_ROWS, 2 * D_MODEL), jnp.float32),
    )


def _unpack_body(in_ref, out_ref):
    t = jnp.transpose(in_ref[...], (1, 0))
    out_ref[0, :, 0:GROUP // 2] = t[0:D_MODEL, :]
    out_ref[0, :, GROUP // 2:GROUP] = t[D_MODEL:2 * D_MODEL, :]


@functools.lru_cache(maxsize=None)
def _make_unpack(B: int, H: int):
    gb = B // GROUP
    return pl.pallas_call(
        _unpack_body,
        grid=(H, gb),
        in_specs=[pl.BlockSpec((GROUP // 2, 2 * D_MODEL),
                               lambda h, bb: (h * gb + bb, 0))],
        out_specs=pl.BlockSpec((1, D_MODEL, GROUP), lambda h, bb: (h, 0, bb)),
        out_shape=jax.ShapeDtypeStruct((H, D_MODEL, B), jnp.float32),
    )


@functools.lru_cache(maxsize=None)
def _make_lookup(B: int):
    assert B % (NUM_WORKERS * CHUNK * NBUF) == 0
    b_per_w = B // NUM_WORKERS
    n_chunks = b_per_w // CHUNK
    n_rounds = n_chunks // NBUF
    mesh = plsc.VectorSubcoreMesh(
        core_axis_name="c", subcore_axis_name="s",
        num_cores=NUM_CORES, num_subcores=NUM_SUBCORES)

    @functools.partial(
        pl.kernel,
        out_type=jax.ShapeDtypeStruct((B, D_MODEL), jnp.float32),
        mesh=mesh,
        scratch_types=[
            pltpu.VMEM((b_per_w,), jnp.int32),
            pltpu.VMEM((b_per_w,), jnp.int32),
            pltpu.VMEM((NBUF, CHUNK, D_MODEL), jnp.float32),
        ] + [pltpu.SemaphoreType.DMA] * (2 * NBUF),
        compiler_params=pltpu.CompilerParams(
            use_tc_tiling_on_sc=False, needs_layout_passes=False),
    )
    def lookup(table_hbm, idx_hbm, out_hbm, idx_v, idx_p, rows_v, *sems):
        sem_g = sems[:NBUF]
        sem_o = sems[NBUF:]
        wid = lax.axis_index("s") * NUM_CORES + lax.axis_index("c")
        base = wid * b_per_w
        pltpu.sync_copy(idx_hbm.at[pl.ds(base, b_per_w)], idx_v)

        lane2 = 2 * lax.iota(jnp.int32, LANES)

        def remap(g, carry):
            for sub in range(2):
                for jc in range(GROUP // 2 // LANES):
                    p = g * GROUP + sub * (GROUP // 2) + jc * LANES
                    v = idx_v[pl.ds(p, LANES)]
                    v = jnp.where(v >= HALF_ROWS,
                                  2 * v - (TABLE_ROWS - 1), 2 * v)
                    pos = (g * GROUP + 2 * jc * LANES + sub) + lane2
                    plsc.store_scatter(idx_p, [pos], v)
            return carry

        lax.fori_loop(0, b_per_w // GROUP, remap, 0)

        def gather_desc(i, b):
            return pltpu.make_async_copy(
                table_hbm.at[idx_p.at[pl.ds(i * CHUNK, CHUNK)]],
                rows_v.at[b], sem_g[b])

        def out_desc(i, b):
            return pltpu.make_async_copy(
                rows_v.at[b], out_hbm.at[pl.ds(base + i * CHUNK, CHUNK)],
                sem_o[b])

        for b in range(NBUF):
            gather_desc(b, b).start()

        def body(r, carry):
            for b in range(NBUF):
                i = r * NBUF + b
                gather_desc(i, b).wait()
                out_desc(i, b).start()
            for b in range(NBUF):
                i = r * NBUF + b
                out_desc(i, b).wait()
                gather_desc(i + NBUF, b).start()
            return carry

        lax.fori_loop(0, n_rounds - 1, body, 0)

        r_last = n_rounds - 1
        for b in range(NBUF):
            i = r_last * NBUF + b
            gather_desc(i, b).wait()
            out_desc(i, b).start()
        for b in range(NBUF):
            out_desc(r_last * NBUF + b, b).wait()

    return lookup


@jax.jit
def kernel(x, weight):
    B, H = x.shape
    V = weight.shape[0]
    flat = jnp.transpose(x).reshape(B * H).astype(jnp.int32)
    wt = jnp.transpose(weight)
    table = _make_repack(V)(wt, wt).reshape(TABLE_ROWS, D_MODEL)
    out_lin = _make_lookup(B * H)(table, flat)
    out_t = _make_unpack(B, H)(out_lin.reshape(B * H // 2, 2 * D_MODEL))
    return jnp.transpose(out_t, (2, 0, 1))


# speedup vs baseline: 2.2919x; 2.0519x over previous
import functools

import jax
import jax.numpy as jnp
from jax import lax
from jax.experimental import pallas as pl
from jax.experimental.pallas import tpu as pltpu
from jax.experimental.pallas import tpu_sc as plsc

D_MODEL = 64
NUM_CORES = 2
NUM_SUBCORES = 16
NUM_WORKERS = NUM_CORES * NUM_SUBCORES
CHUNK = 256
NBUF = 4
LANES = 16
GROUP = 512

REPACK_COLS = 2048
REPACK_GRID = 245
HALF_ROWS = REPACK_COLS * REPACK_GRID
TABLE_ROWS = 2 * HALF_ROWS


def _repack_body(left_ref, right_ref, out_ref):
    lr = jnp.concatenate([left_ref[...], right_ref[...]], axis=0)
    out_ref[...] = jnp.transpose(lr, (1, 0))


@functools.lru_cache(maxsize=None)
def _make_repack(V: int):
    return pl.pallas_call(
        _repack_body,
        grid=(REPACK_GRID,),
        in_specs=[
            pl.BlockSpec((D_MODEL, REPACK_COLS), lambda i: (0, i)),
            pl.BlockSpec((D_MODEL, REPACK_COLS),
                         lambda i: (0, jnp.minimum(i + REPACK_GRID,
                                                   (V - 1) // REPACK_COLS))),
        ],
        out_specs=pl.BlockSpec((REPACK_COLS, 2 * D_MODEL), lambda i: (i, 0)),
        out_shape=jax.ShapeDtypeStruct((HALF_ROWS, 2 * D_MODEL), jnp.float32),
    )


GPB = 8


def _unpack_body(in_ref, out_ref):
    t = jnp.transpose(in_ref[...], (1, 0))
    for g in range(GPB):
        out_ref[0, :, g * GROUP:g * GROUP + GROUP // 2] = (
            t[0:D_MODEL, g * (GROUP // 2):(g + 1) * (GROUP // 2)])
        out_ref[0, :, g * GROUP + GROUP // 2:(g + 1) * GROUP] = (
            t[D_MODEL:2 * D_MODEL, g * (GROUP // 2):(g + 1) * (GROUP // 2)])


@functools.lru_cache(maxsize=None)
def _make_unpack(B: int, H: int):
    span = GROUP * GPB
    gb = B // span
    return pl.pallas_call(
        _unpack_body,
        grid=(H, gb),
        in_specs=[pl.BlockSpec((span // 2, 2 * D_MODEL),
                               lambda h, bb: (h * gb + bb, 0))],
        out_specs=pl.BlockSpec((1, D_MODEL, span), lambda h, bb: (h, 0, bb)),
        out_shape=jax.ShapeDtypeStruct((H, D_MODEL, B), jnp.float32),
    )


@functools.lru_cache(maxsize=None)
def _make_lookup(B: int):
    assert B % (NUM_WORKERS * CHUNK * NBUF) == 0
    b_per_w = B // NUM_WORKERS
    n_chunks = b_per_w // CHUNK
    n_rounds = n_chunks // NBUF
    mesh = plsc.VectorSubcoreMesh(
        core_axis_name="c", subcore_axis_name="s",
        num_cores=NUM_CORES, num_subcores=NUM_SUBCORES)

    @functools.partial(
        pl.kernel,
        out_type=jax.ShapeDtypeStruct((B, D_MODEL), jnp.float32),
        mesh=mesh,
        scratch_types=[
            pltpu.VMEM((b_per_w,), jnp.int32),
            pltpu.VMEM((b_per_w,), jnp.int32),
            pltpu.VMEM((NBUF, CHUNK, D_MODEL), jnp.float32),
        ] + [pltpu.SemaphoreType.DMA] * (2 * NBUF),
        compiler_params=pltpu.CompilerParams(
            use_tc_tiling_on_sc=False, needs_layout_passes=False),
    )
    def lookup(table_hbm, idx_hbm, out_hbm, idx_v, idx_p, rows_v, *sems):
        sem_g = sems[:NBUF]
        sem_o = sems[NBUF:]
        wid = lax.axis_index("s") * NUM_CORES + lax.axis_index("c")
        base = wid * b_per_w
        pltpu.sync_copy(idx_hbm.at[pl.ds(base, b_per_w)], idx_v)

        lane2 = 2 * lax.iota(jnp.int32, LANES)

        def remap(g, carry):
            for sub in range(2):
                for jc in range(GROUP // 2 // LANES):
                    p = g * GROUP + sub * (GROUP // 2) + jc * LANES
                    v = idx_v[pl.ds(p, LANES)]
                    v = jnp.where(v >= HALF_ROWS,
                                  2 * v - (TABLE_ROWS - 1), 2 * v)
                    pos = (g * GROUP + 2 * jc * LANES + sub) + lane2
                    plsc.store_scatter(idx_p, [pos], v)
            return carry

        lax.fori_loop(0, b_per_w // GROUP, remap, 0)

        def gather_desc(i, b):
            return pltpu.make_async_copy(
                table_hbm.at[idx_p.at[pl.ds(i * CHUNK, CHUNK)]],
                rows_v.at[b], sem_g[b])

        def out_desc(i, b):
            return pltpu.make_async_copy(
                rows_v.at[b], out_hbm.at[pl.ds(base + i * CHUNK, CHUNK)],
                sem_o[b])

        for b in range(NBUF):
            gather_desc(b, b).start()

        def body(r, carry):
            for b in range(NBUF):
                i = r * NBUF + b
                gather_desc(i, b).wait()
                out_desc(i, b).start()
            for b in range(NBUF):
                i = r * NBUF + b
                out_desc(i, b).wait()
                gather_desc(i + NBUF, b).start()
            return carry

        lax.fori_loop(0, n_rounds - 1, body, 0)

        r_last = n_rounds - 1
        for b in range(NBUF):
            i = r_last * NBUF + b
            gather_desc(i, b).wait()
            out_desc(i, b).start()
        for b in range(NBUF):
            out_desc(r_last * NBUF + b, b).wait()

    return lookup


@jax.jit
def kernel(x, weight):
    B, H = x.shape
    V = weight.shape[0]
    flat = jnp.transpose(x).reshape(B * H).astype(jnp.int32)
    wt = jnp.transpose(weight)
    table = _make_repack(V)(wt, wt).reshape(TABLE_ROWS, D_MODEL)
    out_lin = _make_lookup(B * H)(table, flat)
    out_t = _make_unpack(B, H)(out_lin.reshape(B * H // 2, 2 * D_MODEL))
    return jnp.transpose(out_t, (2, 0, 1))


# speedup vs baseline: 2.7100x; 1.1825x over previous
import functools

import jax
import jax.numpy as jnp
from jax import lax
from jax.experimental import pallas as pl
from jax.experimental.pallas import tpu as pltpu
from jax.experimental.pallas import tpu_sc as plsc

D_MODEL = 64
NUM_CORES = 2
NUM_SUBCORES = 16
NUM_WORKERS = NUM_CORES * NUM_SUBCORES
CHUNK = 256
NBUF = 4
LANES = 16
GROUP = 512

REPACK_COLS = 8192
REPACK_GRID = 62
HALF_ROWS = REPACK_COLS * REPACK_GRID
TABLE_ROWS = 2 * HALF_ROWS


def _repack_body(left_ref, right_ref, out_ref):
    lr = jnp.concatenate([left_ref[...], right_ref[...]], axis=0)
    out_ref[...] = jnp.transpose(lr, (1, 0))


@functools.lru_cache(maxsize=None)
def _make_repack(V: int):
    return pl.pallas_call(
        _repack_body,
        grid=(REPACK_GRID,),
        in_specs=[
            pl.BlockSpec((D_MODEL, REPACK_COLS), lambda i: (0, i)),
            pl.BlockSpec((D_MODEL, REPACK_COLS),
                         lambda i: (0, jnp.minimum(i + REPACK_GRID,
                                                   (V - 1) // REPACK_COLS))),
        ],
        out_specs=pl.BlockSpec((REPACK_COLS, 2 * D_MODEL), lambda i: (i, 0)),
        out_shape=jax.ShapeDtypeStruct((HALF_ROWS, 2 * D_MODEL), jnp.float32),
    )


GPB = 8


def _unpack_body(in_ref, out_ref):
    t = jnp.transpose(in_ref[...], (1, 0))
    for g in range(GPB):
        out_ref[0, :, g * GROUP:g * GROUP + GROUP // 2] = (
            t[0:D_MODEL, g * (GROUP // 2):(g + 1) * (GROUP // 2)])
        out_ref[0, :, g * GROUP + GROUP // 2:(g + 1) * GROUP] = (
            t[D_MODEL:2 * D_MODEL, g * (GROUP // 2):(g + 1) * (GROUP // 2)])


@functools.lru_cache(maxsize=None)
def _make_unpack(B: int, H: int):
    span = GROUP * GPB
    gb = B // span
    return pl.pallas_call(
        _unpack_body,
        grid=(H, gb),
        in_specs=[pl.BlockSpec((span // 2, 2 * D_MODEL),
                               lambda h, bb: (h * gb + bb, 0))],
        out_specs=pl.BlockSpec((1, D_MODEL, span), lambda h, bb: (h, 0, bb)),
        out_shape=jax.ShapeDtypeStruct((H, D_MODEL, B), jnp.float32),
    )


@functools.lru_cache(maxsize=None)
def _make_lookup(B: int):
    assert B % (NUM_WORKERS * CHUNK * NBUF) == 0
    b_per_w = B // NUM_WORKERS
    n_chunks = b_per_w // CHUNK
    n_rounds = n_chunks // NBUF
    mesh = plsc.VectorSubcoreMesh(
        core_axis_name="c", subcore_axis_name="s",
        num_cores=NUM_CORES, num_subcores=NUM_SUBCORES)

    @functools.partial(
        pl.kernel,
        out_type=jax.ShapeDtypeStruct((B, D_MODEL), jnp.float32),
        mesh=mesh,
        scratch_types=[
            pltpu.VMEM((b_per_w,), jnp.int32),
            pltpu.VMEM((b_per_w,), jnp.int32),
            pltpu.VMEM((NBUF, CHUNK, D_MODEL), jnp.float32),
        ] + [pltpu.SemaphoreType.DMA] * (2 * NBUF),
        compiler_params=pltpu.CompilerParams(
            use_tc_tiling_on_sc=False, needs_layout_passes=False),
    )
    def lookup(table_hbm, idx_hbm, out_hbm, idx_v, idx_p, rows_v, *sems):
        sem_g = sems[:NBUF]
        sem_o = sems[NBUF:]
        wid = lax.axis_index("s") * NUM_CORES + lax.axis_index("c")
        base = wid * b_per_w
        pltpu.sync_copy(idx_hbm.at[pl.ds(base, b_per_w)], idx_v)

        lane2 = 2 * lax.iota(jnp.int32, LANES)

        def remap(g, carry):
            for sub in range(2):
                for jc in range(GROUP // 2 // LANES):
                    p = g * GROUP + sub * (GROUP // 2) + jc * LANES
                    v = idx_v[pl.ds(p, LANES)]
                    v = jnp.where(v >= HALF_ROWS,
                                  2 * v - (TABLE_ROWS - 1), 2 * v)
                    pos = (g * GROUP + 2 * jc * LANES + sub) + lane2
                    plsc.store_scatter(idx_p, [pos], v)
            return carry

        lax.fori_loop(0, b_per_w // GROUP, remap, 0)

        def gather_desc(i, b):
            return pltpu.make_async_copy(
                table_hbm.at[idx_p.at[pl.ds(i * CHUNK, CHUNK)]],
                rows_v.at[b], sem_g[b])

        def out_desc(i, b):
            return pltpu.make_async_copy(
                rows_v.at[b], out_hbm.at[pl.ds(base + i * CHUNK, CHUNK)],
                sem_o[b])

        for b in range(NBUF):
            gather_desc(b, b).start()

        def body(r, carry):
            for b in range(NBUF):
                i = r * NBUF + b
                gather_desc(i, b).wait()
                out_desc(i, b).start()
            for b in range(NBUF):
                i = r * NBUF + b
                out_desc(i, b).wait()
                gather_desc(i + NBUF, b).start()
            return carry

        lax.fori_loop(0, n_rounds - 1, body, 0)

        r_last = n_rounds - 1
        for b in range(NBUF):
            i = r_last * NBUF + b
            gather_desc(i, b).wait()
            out_desc(i, b).start()
        for b in range(NBUF):
            out_desc(r_last * NBUF + b, b).wait()

    return lookup


@jax.jit
def kernel(x, weight):
    B, H = x.shape
    V = weight.shape[0]
    flat = jnp.transpose(x).reshape(B * H).astype(jnp.int32)
    wt = jnp.transpose(weight)
    table = _make_repack(V)(wt, wt).reshape(TABLE_ROWS, D_MODEL)
    out_lin = _make_lookup(B * H)(table, flat)
    out_t = _make_unpack(B, H)(out_lin.reshape(B * H // 2, 2 * D_MODEL))
    return jnp.transpose(out_t, (2, 0, 1))


# speedup vs baseline: 3.0665x; 1.1315x over previous
import functools

import jax
import jax.numpy as jnp
from jax import lax
from jax.experimental import pallas as pl
from jax.experimental.pallas import tpu as pltpu
from jax.experimental.pallas import tpu_sc as plsc

D_MODEL = 64
NUM_CORES = 2
NUM_SUBCORES = 16
NUM_WORKERS = NUM_CORES * NUM_SUBCORES
CHUNK = 256
NBUF = 4
LANES = 16
GROUP = 512

REPACK_COLS = 16384
REPACK_GRID = 31
HALF_ROWS = REPACK_COLS * REPACK_GRID
TABLE_ROWS = 2 * HALF_ROWS


def _repack_body(left_ref, right_ref, out_ref):
    lr = jnp.concatenate([left_ref[...], right_ref[...]], axis=0)
    out_ref[...] = jnp.transpose(lr, (1, 0))


@functools.lru_cache(maxsize=None)
def _make_repack(V: int):
    return pl.pallas_call(
        _repack_body,
        grid=(REPACK_GRID,),
        in_specs=[
            pl.BlockSpec((D_MODEL, REPACK_COLS), lambda i: (0, i)),
            pl.BlockSpec((D_MODEL, REPACK_COLS),
                         lambda i: (0, jnp.minimum(i + REPACK_GRID,
                                                   (V - 1) // REPACK_COLS))),
        ],
        out_specs=pl.BlockSpec((REPACK_COLS, 2 * D_MODEL), lambda i: (i, 0)),
        out_shape=jax.ShapeDtypeStruct((HALF_ROWS, 2 * D_MODEL), jnp.float32),
    )


GPB = 16


def _unpack_body(in_ref, out_ref):
    t = jnp.transpose(in_ref[...], (1, 0))
    for g in range(GPB):
        out_ref[0, :, g * GROUP:g * GROUP + GROUP // 2] = (
            t[0:D_MODEL, g * (GROUP // 2):(g + 1) * (GROUP // 2)])
        out_ref[0, :, g * GROUP + GROUP // 2:(g + 1) * GROUP] = (
            t[D_MODEL:2 * D_MODEL, g * (GROUP // 2):(g + 1) * (GROUP // 2)])


@functools.lru_cache(maxsize=None)
def _make_unpack(B: int, H: int):
    span = GROUP * GPB
    gb = B // span
    return pl.pallas_call(
        _unpack_body,
        grid=(H, gb),
        in_specs=[pl.BlockSpec((span // 2, 2 * D_MODEL),
                               lambda h, bb: (h * gb + bb, 0))],
        out_specs=pl.BlockSpec((1, D_MODEL, span), lambda h, bb: (h, 0, bb)),
        out_shape=jax.ShapeDtypeStruct((H, D_MODEL, B), jnp.float32),
    )


@functools.lru_cache(maxsize=None)
def _make_lookup(B: int):
    assert B % (NUM_WORKERS * CHUNK * NBUF) == 0
    b_per_w = B // NUM_WORKERS
    n_chunks = b_per_w // CHUNK
    n_rounds = n_chunks // NBUF
    mesh = plsc.VectorSubcoreMesh(
        core_axis_name="c", subcore_axis_name="s",
        num_cores=NUM_CORES, num_subcores=NUM_SUBCORES)

    @functools.partial(
        pl.kernel,
        out_type=jax.ShapeDtypeStruct((B, D_MODEL), jnp.float32),
        mesh=mesh,
        scratch_types=[
            pltpu.VMEM((b_per_w,), jnp.int32),
            pltpu.VMEM((b_per_w,), jnp.int32),
            pltpu.VMEM((NBUF, CHUNK, D_MODEL), jnp.float32),
        ] + [pltpu.SemaphoreType.DMA] * (2 * NBUF),
        compiler_params=pltpu.CompilerParams(
            use_tc_tiling_on_sc=False, needs_layout_passes=False),
    )
    def lookup(table_hbm, idx_hbm, out_hbm, idx_v, idx_p, rows_v, *sems):
        sem_g = sems[:NBUF]
        sem_o = sems[NBUF:]
        wid = lax.axis_index("s") * NUM_CORES + lax.axis_index("c")
        base = wid * b_per_w
        pltpu.sync_copy(idx_hbm.at[pl.ds(base, b_per_w)], idx_v)

        lane2 = 2 * lax.iota(jnp.int32, LANES)

        def remap(g, carry):
            for sub in range(2):
                for jc in range(GROUP // 2 // LANES):
                    p = g * GROUP + sub * (GROUP // 2) + jc * LANES
                    v = idx_v[pl.ds(p, LANES)]
                    v = jnp.where(v >= HALF_ROWS,
                                  2 * v - (TABLE_ROWS - 1), 2 * v)
                    pos = (g * GROUP + 2 * jc * LANES + sub) + lane2
                    plsc.store_scatter(idx_p, [pos], v)
            return carry

        lax.fori_loop(0, b_per_w // GROUP, remap, 0)

        def gather_desc(i, b):
            return pltpu.make_async_copy(
                table_hbm.at[idx_p.at[pl.ds(i * CHUNK, CHUNK)]],
                rows_v.at[b], sem_g[b])

        def out_desc(i, b):
            return pltpu.make_async_copy(
                rows_v.at[b], out_hbm.at[pl.ds(base + i * CHUNK, CHUNK)],
                sem_o[b])

        for b in range(NBUF):
            gather_desc(b, b).start()

        def body(r, carry):
            for b in range(NBUF):
                i = r * NBUF + b
                gather_desc(i, b).wait()
                out_desc(i, b).start()
            for b in range(NBUF):
                i = r * NBUF + b
                out_desc(i, b).wait()
                gather_desc(i + NBUF, b).start()
            return carry

        lax.fori_loop(0, n_rounds - 1, body, 0)

        r_last = n_rounds - 1
        for b in range(NBUF):
            i = r_last * NBUF + b
            gather_desc(i, b).wait()
            out_desc(i, b).start()
        for b in range(NBUF):
            out_desc(r_last * NBUF + b, b).wait()

    return lookup


@jax.jit
def kernel(x, weight):
    B, H = x.shape
    V = weight.shape[0]
    flat = jnp.transpose(x).reshape(B * H).astype(jnp.int32)
    wt = jnp.transpose(weight)
    table = _make_repack(V)(wt, wt).reshape(TABLE_ROWS, D_MODEL)
    out_lin = _make_lookup(B * H)(table, flat)
    out_t = _make_unpack(B, H)(out_lin.reshape(B * H // 2, 2 * D_MODEL))
    return jnp.transpose(out_t, (2, 0, 1))
